# baseline (speedup 1.0000x reference)
import functools

import jax
import jax.numpy as jnp
from jax import lax
from jax.experimental import pallas as pl
from jax.experimental.pallas import tpu as pltpu
from jax.experimental.pallas import tpu_sc as plsc

NUM_CORES = 2
NUM_SUBCORES = 16
NW = NUM_CORES * NUM_SUBCORES


@functools.partial(jax.jit, static_argnums=(0, 1, 2))
def _sc_gather4(B, D, interpret, uids, iids, gu_t, gi_t, mu_t, mi_t):
    b_per_w = B // NW
    mesh = plsc.VectorSubcoreMesh(core_axis_name="c", subcore_axis_name="s")

    @functools.partial(
        pl.kernel,
        out_type=[jax.ShapeDtypeStruct((B, D), jnp.float32)] * 4,
        mesh=mesh,
        interpret=interpret,
        compiler_params=pltpu.CompilerParams(use_tc_tiling_on_sc=False),
        scratch_types=[
            pltpu.VMEM((b_per_w,), jnp.int32),
            pltpu.VMEM((b_per_w,), jnp.int32),
            pltpu.VMEM((b_per_w, D), jnp.float32),
            pltpu.VMEM((b_per_w, D), jnp.float32),
            pltpu.VMEM((b_per_w, D), jnp.float32),
            pltpu.VMEM((b_per_w, D), jnp.float32),
            pltpu.SemaphoreType.DMA,
        ],
    )
    def gather(uids_h, iids_h, gu_h, gi_h, mu_h, mi_h,
               gu_o, gi_o, mu_o, mi_o,
               uidx, iidx, gu, gi, mu, mi, sem):
        wid = lax.axis_index("s") * NUM_CORES + lax.axis_index("c")
        base = wid * b_per_w
        pltpu.sync_copy(uids_h.at[pl.ds(base, b_per_w)], uidx)
        pltpu.sync_copy(iids_h.at[pl.ds(base, b_per_w)], iidx)
        c1 = pltpu.async_copy(gu_h.at[uidx], gu, sem)
        c2 = pltpu.async_copy(gi_h.at[iidx], gi, sem)
        c3 = pltpu.async_copy(mu_h.at[uidx], mu, sem)
        c4 = pltpu.async_copy(mi_h.at[iidx], mi, sem)
        c1.wait()
        c2.wait()
        c3.wait()
        c4.wait()
        pltpu.sync_copy(gu, gu_o.at[pl.ds(base, b_per_w)])
        pltpu.sync_copy(gi, gi_o.at[pl.ds(base, b_per_w)])
        pltpu.sync_copy(mu, mu_o.at[pl.ds(base, b_per_w)])
        pltpu.sync_copy(mi, mi_o.at[pl.ds(base, b_per_w)])

    return gather(uids, iids, gu_t, gi_t, mu_t, mi_t)


def _tc_body(gu_r, gi_r, mu_r, mi_r, w1a_r, w1b_r, b1_r, w2_r, b2_r,
             w3_r, b3_r, wog_r, wom_r, bo_r, out_r):
    f32 = jnp.float32
    prod = gu_r[...] * gi_r[...]
    h = (jnp.dot(mu_r[...], w1a_r[...], preferred_element_type=f32)
         + jnp.dot(mi_r[...], w1b_r[...], preferred_element_type=f32)
         + b1_r[...])
    h = jnp.where(h >= 0, h, 0.01 * h)
    h = jnp.dot(h, w2_r[...], preferred_element_type=f32) + b2_r[...]
    h = jnp.where(h >= 0, h, 0.01 * h)
    h = jnp.dot(h, w3_r[...], preferred_element_type=f32) + b3_r[...]
    out_r[...] = (jnp.dot(prod, wog_r[...], preferred_element_type=f32)
                  + jnp.dot(h, wom_r[...], preferred_element_type=f32)
                  + bo_r[...])


@functools.partial(jax.jit, static_argnums=(0,))
def _tc_mlp(interpret, gu, gi, mu, mi, W1a, W1b, b1, W2, b2, W3, b3,
            Wog, Wom, bo):
    B, D = gu.shape
    blk = 2048
    grid = (B // blk,)
    data_spec = pl.BlockSpec((blk, D), lambda i: (i, 0))

    def wspec(a):
        return pl.BlockSpec(a.shape, lambda i: (0,) * a.ndim)

    return pl.pallas_call(
        _tc_body,
        grid=grid,
        in_specs=[data_spec, data_spec, data_spec, data_spec,
                  wspec(W1a), wspec(W1b), wspec(b1), wspec(W2), wspec(b2),
                  wspec(W3), wspec(b3), wspec(Wog), wspec(Wom), wspec(bo)],
        out_specs=pl.BlockSpec((blk, 1), lambda i: (i, 0)),
        out_shape=jax.ShapeDtypeStruct((B, 1), jnp.float32),
        interpret=interpret,
    )(gu, gi, mu, mi, W1a, W1b, b1, W2, b2, W3, b3, Wog, Wom, bo)


def kernel(user_ids, item_ids, gmf_user, gmf_item, mlp_user, mlp_item,
           W1, b1, W2, b2, W3, b3, Wo, bo, *, interpret_tc=False):
    B = user_ids.shape[0]
    D = gmf_user.shape[1]
    uids = user_ids.astype(jnp.int32)
    iids = item_ids.astype(jnp.int32)
    gu, gi, mu, mi = _sc_gather4(B, D, False, uids, iids,
                                 gmf_user, gmf_item, mlp_user, mlp_item)
    W1a, W1b = W1[:D], W1[D:]
    Wog, Wom = Wo[:D], Wo[D:]
    return _tc_mlp(interpret_tc, gu, gi, mu, mi, W1a, W1b,
                   b1.reshape(1, -1), W2, b2.reshape(1, -1),
                   W3, b3.reshape(1, -1), Wog, Wom, bo.reshape(1, 1))


# speedup vs baseline: 1.0022x; 1.0022x over previous
import functools

import jax
import jax.numpy as jnp
from jax import lax
from jax.experimental import pallas as pl
from jax.experimental.pallas import tpu as pltpu
from jax.experimental.pallas import tpu_sc as plsc

NUM_CORES = 2
NUM_SUBCORES = 16
NW = NUM_CORES * NUM_SUBCORES
L = 16
CH = 128


@functools.partial(jax.jit, static_argnums=(0, 1))
def _sc_gather_pack(B, D, uids, iids, gu_t, gi_t, mu_t, mi_t):
    b_per_w = B // NW
    n_chunks = b_per_w // CH
    mesh = plsc.VectorSubcoreMesh(core_axis_name="c", subcore_axis_name="s")

    @functools.partial(
        pl.kernel,
        out_type=jax.ShapeDtypeStruct((B, 4 * D), jnp.float32),
        mesh=mesh,
        scratch_types=[
            pltpu.VMEM((n_chunks, CH), jnp.int32),
            pltpu.VMEM((n_chunks, CH), jnp.int32),
            pltpu.VMEM((n_chunks, CH), jnp.int32),
            pltpu.VMEM((n_chunks, CH), jnp.int32),
            pltpu.VMEM((CH, 4 * D), jnp.float32),
            pltpu.VMEM((CH, 4 * D), jnp.float32),
            pltpu.VMEM((CH, 4 * D), jnp.float32),
            pltpu.VMEM((CH, 4 * D), jnp.float32),
            pltpu.VMEM((CH, 4 * D), jnp.float32),
            pltpu.SemaphoreType.DMA,
        ],
    )
    def gather(uids_h, iids_h, gu_h, gi_h, mu_h, mi_h, out_h,
               bu, bi, ou, oi, rgu, rgi, rmu, rmi, outb, sem):
        wid = lax.axis_index("s") * NUM_CORES + lax.axis_index("c")
        base = wid * b_per_w

        for k in range(n_chunks):
            pltpu.sync_copy(uids_h.at[pl.ds(base + k * CH, CH)], bu.at[k])
            pltpu.sync_copy(iids_h.at[pl.ds(base + k * CH, CH)], bi.at[k])

        def to_big_off(i, _):
            k = i // (CH // L)
            j = (i % (CH // L)) * L
            u = bu[k, pl.ds(j, L)]
            v = bi[k, pl.ds(j, L)]
            ou[k, pl.ds(j, L)] = (u & 3) << 5
            oi[k, pl.ds(j, L)] = (v & 3) << 5
            bu[k, pl.ds(j, L)] = u >> 2
            bi[k, pl.ds(j, L)] = v >> 2
            return _

        lax.fori_loop(0, n_chunks * (CH // L), to_big_off, 0)

        def extract(g, k):
            ouv = ou[k, pl.ds(g * L, L)]
            oiv = oi[k, pl.ds(g * L, L)]
            for lane in range(L):
                s = g * L + lane
                o_u = ouv[lane]
                o_i = oiv[lane]
                gu0 = rgu[s, pl.ds(o_u, L)]
                gu1 = rgu[s, pl.ds(o_u + L, L)]
                gi0 = rgi[s, pl.ds(o_i, L)]
                gi1 = rgi[s, pl.ds(o_i + L, L)]
                outb[s, pl.ds(0, L)] = gu0 * gi0
                outb[s, pl.ds(L, L)] = gu1 * gi1
                outb[s, pl.ds(2 * L, L)] = rmu[s, pl.ds(o_u, L)]
                outb[s, pl.ds(3 * L, L)] = rmu[s, pl.ds(o_u + L, L)]
                outb[s, pl.ds(4 * L, L)] = rmi[s, pl.ds(o_i, L)]
                outb[s, pl.ds(5 * L, L)] = rmi[s, pl.ds(o_i + L, L)]
            return k

        for k in range(n_chunks):
            c1 = pltpu.async_copy(gu_h.at[bu.at[k]], rgu, sem)
            c2 = pltpu.async_copy(gi_h.at[bi.at[k]], rgi, sem)
            c3 = pltpu.async_copy(mu_h.at[bu.at[k]], rmu, sem)
            c4 = pltpu.async_copy(mi_h.at[bi.at[k]], rmi, sem)
            c1.wait()
            c2.wait()
            c3.wait()
            c4.wait()
            lax.fori_loop(0, CH // L, extract, k)
            pltpu.sync_copy(outb, out_h.at[pl.ds(base + k * CH, CH)])

    return gather(uids, iids, gu_t, gi_t, mu_t, mi_t)


def _tc_body(x_r, w1a_r, w1b_r, b1_r, w2_r, b2_r, w3_r, b3_r,
             wog_r, wom_r, bo_r, out_r):
    f32 = jnp.float32
    x = x_r[...]
    prod, mu, mi = x[:, 0:32], x[:, 32:64], x[:, 64:96]
    h = (jnp.dot(mu, w1a_r[...], preferred_element_type=f32)
         + jnp.dot(mi, w1b_r[...], preferred_element_type=f32)
         + b1_r[...])
    h = jnp.where(h >= 0, h, 0.01 * h)
    h = jnp.dot(h, w2_r[...], preferred_element_type=f32) + b2_r[...]
    h = jnp.where(h >= 0, h, 0.01 * h)
    h = jnp.dot(h, w3_r[...], preferred_element_type=f32) + b3_r[...]
    out_r[...] = (jnp.dot(prod, wog_r[...], preferred_element_type=f32)
                  + jnp.dot(h, wom_r[...], preferred_element_type=f32)
                  + bo_r[...])


@jax.jit
def _tc_mlp(packed, W1a, W1b, b1, W2, b2, W3, b3, Wog, Wom, bo):
    B, P = packed.shape
    blk = 2048
    grid = (B // blk,)

    def wspec(a):
        return pl.BlockSpec(a.shape, lambda i: (0,) * a.ndim)

    return pl.pallas_call(
        _tc_body,
        grid=grid,
        in_specs=[pl.BlockSpec((blk, P), lambda i: (i, 0)),
                  wspec(W1a), wspec(W1b), wspec(b1), wspec(W2), wspec(b2),
                  wspec(W3), wspec(b3), wspec(Wog), wspec(Wom), wspec(bo)],
        out_specs=pl.BlockSpec((blk, 1), lambda i: (i, 0)),
        out_shape=jax.ShapeDtypeStruct((B, 1), jnp.float32),
    )(packed, W1a, W1b, b1, W2, b2, W3, b3, Wog, Wom, bo)


def kernel(user_ids, item_ids, gmf_user, gmf_item, mlp_user, mlp_item,
           W1, b1, W2, b2, W3, b3, Wo, bo):
    B = user_ids.shape[0]
    V, D = gmf_user.shape
    uids = user_ids.astype(jnp.int32)
    iids = item_ids.astype(jnp.int32)
    packed = _sc_gather_pack(B, D, uids, iids,
                             gmf_user.reshape(V // 4, 4 * D),
                             gmf_item.reshape(V // 4, 4 * D),
                             mlp_user.reshape(V // 4, 4 * D),
                             mlp_item.reshape(V // 4, 4 * D))
    W1a, W1b = W1[:D], W1[D:]
    Wog, Wom = Wo[:D], Wo[D:]
    return _tc_mlp(packed, W1a, W1b, b1.reshape(1, -1), W2,
                   b2.reshape(1, -1), W3, b3.reshape(1, -1),
                   Wog, Wom, bo.reshape(1, 1))


# speedup vs baseline: 1.4239x; 1.4208x over previous
import functools

import jax
import jax.numpy as jnp
from jax import lax
from jax.experimental import pallas as pl
from jax.experimental.pallas import tpu as pltpu
from jax.experimental.pallas import tpu_sc as plsc

NUM_CORES = 2
NUM_SUBCORES = 16
NW = NUM_CORES * NUM_SUBCORES
L = 16
CH = 128


@functools.partial(jax.jit, static_argnums=(0, 1))
def _sc_gather_pack(B, D, uids, iids, gu_t, gi_t, mu_t, mi_t):
    b_per_w = B // NW
    n_chunks = b_per_w // CH
    mesh = plsc.VectorSubcoreMesh(core_axis_name="c", subcore_axis_name="s")

    @functools.partial(
        pl.kernel,
        out_type=jax.ShapeDtypeStruct((B, 4 * D), jnp.float32),
        mesh=mesh,
        scratch_types=[
            pltpu.VMEM((b_per_w,), jnp.int32),
            pltpu.VMEM((b_per_w,), jnp.int32),
            pltpu.VMEM((CH, D), jnp.float32),
            pltpu.VMEM((CH, D), jnp.float32),
            pltpu.VMEM((CH, D), jnp.float32),
            pltpu.VMEM((CH, D), jnp.float32),
            pltpu.VMEM((CH, 4 * D), jnp.float32),
            pltpu.SemaphoreType.DMA,
        ],
    )
    def gather(uids_h, iids_h, gu_h, gi_h, mu_h, mi_h, out_h,
               uidx, iidx, gu, gi, mu, mi, outb, sem):
        wid = lax.axis_index("s") * NUM_CORES + lax.axis_index("c")
        base = wid * b_per_w
        pltpu.sync_copy(uids_h.at[pl.ds(base, b_per_w)], uidx)
        pltpu.sync_copy(iids_h.at[pl.ds(base, b_per_w)], iidx)

        def fetch_group(g, k):
            uvec = uidx[pl.ds(k * CH + g * L, L)]
            ivec = iidx[pl.ds(k * CH + g * L, L)]
            for lane in range(L):
                s = g * L + lane
                ru = uvec[lane]
                ri = ivec[lane]
                pltpu.async_copy(gu_h.at[pl.ds(ru, 1)], gu.at[pl.ds(s, 1)], sem)
                pltpu.async_copy(mu_h.at[pl.ds(ru, 1)], mu.at[pl.ds(s, 1)], sem)
                pltpu.async_copy(gi_h.at[pl.ds(ri, 1)], gi.at[pl.ds(s, 1)], sem)
                pltpu.async_copy(mi_h.at[pl.ds(ri, 1)], mi.at[pl.ds(s, 1)], sem)
            return k

        def pack_row(s, _):
            a0 = gu[s, pl.ds(0, L)] * gi[s, pl.ds(0, L)]
            a1 = gu[s, pl.ds(L, L)] * gi[s, pl.ds(L, L)]
            outb[s, pl.ds(0, L)] = a0
            outb[s, pl.ds(L, L)] = a1
            outb[s, pl.ds(2 * L, L)] = mu[s, pl.ds(0, L)]
            outb[s, pl.ds(3 * L, L)] = mu[s, pl.ds(L, L)]
            outb[s, pl.ds(4 * L, L)] = mi[s, pl.ds(0, L)]
            outb[s, pl.ds(5 * L, L)] = mi[s, pl.ds(L, L)]
            return _

        for k in range(n_chunks):
            lax.fori_loop(0, CH // L, fetch_group, k)
            for buf in (gu, gi, mu, mi):
                pltpu.make_async_copy(gu_h.at[pl.ds(0, CH)], buf, sem).wait()
            lax.fori_loop(0, CH, pack_row, 0)
            pltpu.sync_copy(outb, out_h.at[pl.ds(base + k * CH, CH)])

    return gather(uids, iids, gu_t, gi_t, mu_t, mi_t)


def _tc_body(x_r, w1a_r, w1b_r, b1_r, w2_r, b2_r, w3_r, b3_r,
             wog_r, wom_r, bo_r, out_r):
    f32 = jnp.float32
    x = x_r[...]
    prod, mu, mi = x[:, 0:32], x[:, 32:64], x[:, 64:96]
    h = (jnp.dot(mu, w1a_r[...], preferred_element_type=f32)
         + jnp.dot(mi, w1b_r[...], preferred_element_type=f32)
         + b1_r[...])
    h = jnp.where(h >= 0, h, 0.01 * h)
    h = jnp.dot(h, w2_r[...], preferred_element_type=f32) + b2_r[...]
    h = jnp.where(h >= 0, h, 0.01 * h)
    h = jnp.dot(h, w3_r[...], preferred_element_type=f32) + b3_r[...]
    out_r[...] = (jnp.dot(prod, wog_r[...], preferred_element_type=f32)
                  + jnp.dot(h, wom_r[...], preferred_element_type=f32)
                  + bo_r[...])


@jax.jit
def _tc_mlp(packed, W1a, W1b, b1, W2, b2, W3, b3, Wog, Wom, bo):
    B, P = packed.shape
    blk = 2048
    grid = (B // blk,)

    def wspec(a):
        return pl.BlockSpec(a.shape, lambda i: (0,) * a.ndim)

    return pl.pallas_call(
        _tc_body,
        grid=grid,
        in_specs=[pl.BlockSpec((blk, P), lambda i: (i, 0)),
                  wspec(W1a), wspec(W1b), wspec(b1), wspec(W2), wspec(b2),
                  wspec(W3), wspec(b3), wspec(Wog), wspec(Wom), wspec(bo)],
        out_specs=pl.BlockSpec((blk, 1), lambda i: (i, 0)),
        out_shape=jax.ShapeDtypeStruct((B, 1), jnp.float32),
    )(packed, W1a, W1b, b1, W2, b2, W3, b3, Wog, Wom, bo)


def kernel(user_ids, item_ids, gmf_user, gmf_item, mlp_user, mlp_item,
           W1, b1, W2, b2, W3, b3, Wo, bo):
    B = user_ids.shape[0]
    D = gmf_user.shape[1]
    uids = user_ids.astype(jnp.int32)
    iids = item_ids.astype(jnp.int32)
    packed = _sc_gather_pack(B, D, uids, iids,
                             gmf_user, gmf_item, mlp_user, mlp_item)
    W1a, W1b = W1[:D], W1[D:]
    Wog, Wom = Wo[:D], Wo[D:]
    return _tc_mlp(packed, W1a, W1b, b1.reshape(1, -1), W2,
                   b2.reshape(1, -1), W3, b3.reshape(1, -1),
                   Wog, Wom, bo.reshape(1, 1))


# speedup vs baseline: 1.4250x; 1.0008x over previous
import functools

import jax
import jax.numpy as jnp
from jax import lax
from jax.experimental import pallas as pl
from jax.experimental.pallas import tpu as pltpu
from jax.experimental.pallas import tpu_sc as plsc

NUM_CORES = 2
NUM_SUBCORES = 16
NW = NUM_CORES * NUM_SUBCORES
L = 16
CH = 128


@functools.partial(jax.jit, static_argnums=(0, 1))
def _sc_gather_pack(B, D, uids, iids, gu_t, gi_t, mu_t, mi_t):
    b_per_w = B // NW
    n_chunks = b_per_w // CH
    mesh = plsc.VectorSubcoreMesh(core_axis_name="c", subcore_axis_name="s")

    @functools.partial(
        pl.kernel,
        out_type=jax.ShapeDtypeStruct((B, 4 * D), jnp.float32),
        mesh=mesh,
        compiler_params=pltpu.CompilerParams(use_tc_tiling_on_sc=True),
        scratch_types=[
            pltpu.VMEM((b_per_w,), jnp.int32),
            pltpu.VMEM((b_per_w,), jnp.int32),
            pltpu.VMEM((CH, D), jnp.float32),
            pltpu.VMEM((CH, D), jnp.float32),
            pltpu.VMEM((CH, D), jnp.float32),
            pltpu.VMEM((CH, D), jnp.float32),
            pltpu.VMEM((CH, 4 * D), jnp.float32),
            pltpu.SemaphoreType.DMA,
        ],
    )
    def gather(uids_h, iids_h, gu_h, gi_h, mu_h, mi_h, out_h,
               uidx, iidx, gu, gi, mu, mi, outb, sem):
        wid = lax.axis_index("s") * NUM_CORES + lax.axis_index("c")
        base = wid * b_per_w
        pltpu.sync_copy(uids_h.at[pl.ds(base, b_per_w)], uidx)
        pltpu.sync_copy(iids_h.at[pl.ds(base, b_per_w)], iidx)

        def fetch_group(g, k):
            uvec = uidx[pl.ds(k * CH + g * L, L)]
            ivec = iidx[pl.ds(k * CH + g * L, L)]
            for lane in range(L):
                s = g * L + lane
                ru = uvec[lane]
                ri = ivec[lane]
                pltpu.async_copy(gu_h.at[pl.ds(ru, 1)], gu.at[pl.ds(s, 1)], sem)
                pltpu.async_copy(mu_h.at[pl.ds(ru, 1)], mu.at[pl.ds(s, 1)], sem)
                pltpu.async_copy(gi_h.at[pl.ds(ri, 1)], gi.at[pl.ds(s, 1)], sem)
                pltpu.async_copy(mi_h.at[pl.ds(ri, 1)], mi.at[pl.ds(s, 1)], sem)
            return k

        def pack_row(s, _):
            a0 = gu[s, pl.ds(0, L)] * gi[s, pl.ds(0, L)]
            a1 = gu[s, pl.ds(L, L)] * gi[s, pl.ds(L, L)]
            outb[s, pl.ds(0, L)] = a0
            outb[s, pl.ds(L, L)] = a1
            outb[s, pl.ds(2 * L, L)] = mu[s, pl.ds(0, L)]
            outb[s, pl.ds(3 * L, L)] = mu[s, pl.ds(L, L)]
            outb[s, pl.ds(4 * L, L)] = mi[s, pl.ds(0, L)]
            outb[s, pl.ds(5 * L, L)] = mi[s, pl.ds(L, L)]
            return _

        for k in range(n_chunks):
            lax.fori_loop(0, CH // L, fetch_group, k)
            for buf in (gu, gi, mu, mi):
                pltpu.make_async_copy(gu_h.at[pl.ds(0, CH)], buf, sem).wait()
            lax.fori_loop(0, CH, pack_row, 0)
            pltpu.sync_copy(outb, out_h.at[pl.ds(base + k * CH, CH)])

    return gather(uids, iids, gu_t, gi_t, mu_t, mi_t)


def _tc_body(x_r, w1a_r, w1b_r, b1_r, w2_r, b2_r, w3_r, b3_r,
             wog_r, wom_r, bo_r, out_r):
    f32 = jnp.float32
    x = x_r[...]
    prod, mu, mi = x[:, 0:32], x[:, 32:64], x[:, 64:96]
    h = (jnp.dot(mu, w1a_r[...], preferred_element_type=f32)
         + jnp.dot(mi, w1b_r[...], preferred_element_type=f32)
         + b1_r[...])
    h = jnp.where(h >= 0, h, 0.01 * h)
    h = jnp.dot(h, w2_r[...], preferred_element_type=f32) + b2_r[...]
    h = jnp.where(h >= 0, h, 0.01 * h)
    h = jnp.dot(h, w3_r[...], preferred_element_type=f32) + b3_r[...]
    out_r[...] = (jnp.dot(prod, wog_r[...], preferred_element_type=f32)
                  + jnp.dot(h, wom_r[...], preferred_element_type=f32)
                  + bo_r[...])


@jax.jit
def _tc_mlp(packed, W1a, W1b, b1, W2, b2, W3, b3, Wog, Wom, bo):
    B, P = packed.shape
    blk = 2048
    grid = (B // blk,)

    def wspec(a):
        return pl.BlockSpec(a.shape, lambda i: (0,) * a.ndim)

    return pl.pallas_call(
        _tc_body,
        grid=grid,
        in_specs=[pl.BlockSpec((blk, P), lambda i: (i, 0)),
                  wspec(W1a), wspec(W1b), wspec(b1), wspec(W2), wspec(b2),
                  wspec(W3), wspec(b3), wspec(Wog), wspec(Wom), wspec(bo)],
        out_specs=pl.BlockSpec((blk, 1), lambda i: (i, 0)),
        out_shape=jax.ShapeDtypeStruct((B, 1), jnp.float32),
    )(packed, W1a, W1b, b1, W2, b2, W3, b3, Wog, Wom, bo)


def kernel(user_ids, item_ids, gmf_user, gmf_item, mlp_user, mlp_item,
           W1, b1, W2, b2, W3, b3, Wo, bo):
    B = user_ids.shape[0]
    D = gmf_user.shape[1]
    uids = user_ids.astype(jnp.int32)
    iids = item_ids.astype(jnp.int32)
    packed = _sc_gather_pack(B, D, uids, iids,
                             gmf_user, gmf_item, mlp_user, mlp_item)
    W1a, W1b = W1[:D], W1[D:]
    Wog, Wom = Wo[:D], Wo[D:]
    return _tc_mlp(packed, W1a, W1b, b1.reshape(1, -1), W2,
                   b2.reshape(1, -1), W3, b3.reshape(1, -1),
                   Wog, Wom, bo.reshape(1, 1))


# speedup vs baseline: 3.4533x; 2.4233x over previous
import functools

import jax
import jax.numpy as jnp
from jax import lax
from jax.experimental import pallas as pl
from jax.experimental.pallas import tpu as pltpu
from jax.experimental.pallas import tpu_sc as plsc

NUM_CORES = 2
NUM_SUBCORES = 16
NW = NUM_CORES * NUM_SUBCORES
L = 16
SEG = 128
RW = 1024
NBUF = 8


@functools.partial(jax.jit, static_argnums=(0, 1))
def _sc_tile_image(D, V, gu_t, gi_t, mu_t, mi_t, gu_e, gi_e, mu_e, mi_e):
    ntc = (V + 127) // 128
    n_full = (V // RW)
    n_it = (n_full + NW - 1) // NW
    tail0 = n_full * RW
    ntr = (D + 7) // 8
    mesh = plsc.VectorSubcoreMesh(core_axis_name="c", subcore_axis_name="s")

    @functools.partial(
        pl.kernel,
        out_type=[jax.ShapeDtypeStruct((ntr * ntc * 8, 128), jnp.float32)] * 4,
        mesh=mesh,
        compiler_params=pltpu.CompilerParams(use_tc_tiling_on_sc=True),
        scratch_types=(
            [pltpu.VMEM((8, RW), jnp.float32)] * NBUF
            + [pltpu.SemaphoreType.DMA] * NBUF
            + [pltpu.SemaphoreType.DMA] * NBUF
            + [pltpu.VMEM((8, 128), jnp.float32), pltpu.SemaphoreType.DMA]
        ),
    )
    def copy_k(gu_h, gi_h, mu_h, mi_h, gu_eh, gi_eh, mu_eh, mi_eh,
               fgu, fgi, fmu, fmi, *scratch):
        bufs = scratch[:NBUF]
        rsem = scratch[NBUF:2 * NBUF]
        wsem = scratch[2 * NBUF:3 * NBUF]
        tbuf, tsem = scratch[3 * NBUF], scratch[3 * NBUF + 1]
        srcs = (gu_h, gi_h, mu_h, mi_h)
        dsts = (fgu, fgi, fmu, fmi)
        wid = lax.axis_index("s") * NUM_CORES + lax.axis_index("c")
        steps = [(t, tr) for t in range(4) for tr in range(ntr)]

        def body(it, carry):
            ch = it * NW + wid

            @pl.when(ch < n_full)
            def _active():
                off = pl.multiple_of(ch * RW, RW)

                def read(s, first_use):
                    t, tr = steps[s]
                    b = s % NBUF
                    drain = pltpu.make_async_copy(
                        srcs[0].at[pl.ds(0, 8), pl.ds(0, RW)],
                        bufs[b], wsem[b])
                    if first_use:
                        @pl.when(it > 0)
                        def _():
                            drain.wait()
                    else:
                        drain.wait()
                    return pltpu.async_copy(
                        srcs[t].at[pl.ds(tr * 8, 8), pl.ds(off, RW)],
                        bufs[b], rsem[b])

                handles = {s: read(s, True) for s in range(NBUF)}
                for s in range(len(steps)):
                    t, tr = steps[s]
                    b = s % NBUF
                    handles[s].wait()
                    r0 = pl.multiple_of(tr * ntc * 8 + (off // 16), 8)
                    for j in range(RW // 128):
                        pltpu.async_copy(
                            bufs[b].at[:, pl.ds(j * 128, 128)],
                            dsts[t].at[pl.ds(r0 + 8 * j, 8)],
                            wsem[b])
                    if s + NBUF < len(steps):
                        handles[s + NBUF] = read(s + NBUF, False)
            return carry

        lax.fori_loop(0, n_it, body, 0)

        for b in range(NBUF):
            pltpu.make_async_copy(srcs[0].at[pl.ds(0, 8), pl.ds(0, RW)],
                                  bufs[b], wsem[b]).wait()

        edge = (gu_eh, gi_eh, mu_eh, mi_eh)
        n_tail_full = (V - tail0) // 128
        n_items = n_tail_full + (1 if V % 128 else 0)
        items = [(t, tr, k) for t in range(4) for tr in range(ntr)
                 for k in range(n_items)]
        for n, (t, tr, k) in enumerate(items):
            @pl.when(wid == n % NW)
            def _tail(t=t, tr=tr, k=k):
                tc = tail0 // 128 + k
                if k < n_tail_full:
                    src = srcs[t].at[pl.ds(tr * 8, 8), pl.ds(tc * 128, 128)]
                else:
                    src = edge[t].at[pl.ds(tr * 8, 8), :]
                pltpu.async_copy(src, tbuf, tsem).wait()
                pltpu.async_copy(
                    tbuf, dsts[t].at[pl.ds((tr * ntc + tc) * 8, 8)],
                    tsem).wait()

    return copy_k(gu_t, gi_t, mu_t, mi_t, gu_e, gi_e, mu_e, mi_e)


@functools.partial(jax.jit, static_argnums=(0, 1, 2))
def _sc_gather_pack(B, D, V, uids, iids, gu_f, gi_f, mu_f, mi_f):
    b_per_w = B // NW
    n_el = b_per_w * D
    n_seg = n_el // SEG
    segs_per_d = b_per_w // SEG
    trow = ((V + 127) // 128) * 1024
    mesh = plsc.VectorSubcoreMesh(core_axis_name="c", subcore_axis_name="s")

    @functools.partial(
        pl.kernel,
        out_type=jax.ShapeDtypeStruct((3 * D, B), jnp.float32),
        mesh=mesh,
        compiler_params=pltpu.CompilerParams(use_tc_tiling_on_sc=False),
        scratch_types=[
            pltpu.VMEM((b_per_w,), jnp.int32),
            pltpu.VMEM((b_per_w,), jnp.int32),
            pltpu.VMEM((n_seg, SEG), jnp.int32),
            pltpu.VMEM((n_seg, SEG), jnp.int32),
            pltpu.VMEM((D, b_per_w), jnp.float32),
            pltpu.VMEM((D, b_per_w), jnp.float32),
            pltpu.VMEM((D, b_per_w), jnp.float32),
            pltpu.VMEM((D, b_per_w), jnp.float32),
            pltpu.SemaphoreType.DMA,
        ],
    )
    def gather(uids_h, iids_h, gu_h, gi_h, mu_h, mi_h, out_h,
               uidx, iidx, uel, iel, gub, gib, mub, mib, sem):
        wid = lax.axis_index("s") * NUM_CORES + lax.axis_index("c")
        base = wid * b_per_w
        pltpu.sync_copy(uids_h.at[pl.ds(base, b_per_w)], uidx)
        pltpu.sync_copy(iids_h.at[pl.ds(base, b_per_w)], iidx)

        def build_idx(i, _):
            seg = i // (SEG // L)
            g = i % (SEG // L)
            s0 = (seg % segs_per_d) * SEG + g * L
            d = seg // segs_per_d
            kd = (d // 8) * trow + (d % 8) * 128
            u = uidx[pl.ds(s0, L)]
            v = iidx[pl.ds(s0, L)]
            uel[seg, pl.ds(g * L, L)] = ((u >> 7) << 10) + (u & 127) + kd
            iel[seg, pl.ds(g * L, L)] = ((v >> 7) << 10) + (v & 127) + kd
            return _

        lax.fori_loop(0, n_seg * (SEG // L), build_idx, 0)

        def fire(seg, _):
            d = seg // segs_per_d
            c0 = (seg % segs_per_d) * SEG
            pltpu.async_copy(gu_h.at[uel.at[seg]],
                             gub.at[d, pl.ds(c0, SEG)], sem)
            pltpu.async_copy(gi_h.at[iel.at[seg]],
                             gib.at[d, pl.ds(c0, SEG)], sem)
            pltpu.async_copy(mu_h.at[uel.at[seg]],
                             mub.at[d, pl.ds(c0, SEG)], sem)
            pltpu.async_copy(mi_h.at[iel.at[seg]],
                             mib.at[d, pl.ds(c0, SEG)], sem)
            return _

        lax.fori_loop(0, n_seg, fire, 0)

        def drain(d, _):
            for buf in (gub, gib, mub, mib):
                pltpu.make_async_copy(gu_h.at[pl.ds(0, b_per_w)],
                                      buf.at[d], sem).wait()
            return _

        lax.fori_loop(0, D, drain, 0)

        def prod(i, _):
            d = i // (b_per_w // L)
            g = i % (b_per_w // L)
            sl = pl.ds(g * L, L)
            gub[d, sl] = gub[d, sl] * gib[d, sl]
            return _

        lax.fori_loop(0, D * (b_per_w // L), prod, 0)

        col = pl.ds(base, b_per_w)
        pltpu.sync_copy(gub, out_h.at[pl.ds(0, D), col])
        pltpu.sync_copy(mub, out_h.at[pl.ds(D, D), col])
        pltpu.sync_copy(mib, out_h.at[pl.ds(2 * D, D), col])

    return gather(uids, iids, gu_f, gi_f, mu_f, mi_f)


def _tc_body(x_r, w1a_r, w1b_r, b1_r, w2_r, b2_r, w3_r, b3_r,
             wog_r, wom_r, bo_r, out_r):
    f32 = jnp.float32
    x = x_r[...]
    prod, mu, mi = x[0:32, :], x[32:64, :], x[64:96, :]
    h = (jnp.dot(w1a_r[...], mu, preferred_element_type=f32)
         + jnp.dot(w1b_r[...], mi, preferred_element_type=f32)
         + b1_r[...])
    h = jnp.where(h >= 0, h, 0.01 * h)
    h = jnp.dot(w2_r[...], h, preferred_element_type=f32) + b2_r[...]
    h = jnp.where(h >= 0, h, 0.01 * h)
    h = jnp.dot(w3_r[...], h, preferred_element_type=f32) + b3_r[...]
    out_r[...] = (jnp.dot(wog_r[...], prod, preferred_element_type=f32)
                  + jnp.dot(wom_r[...], h, preferred_element_type=f32)
                  + bo_r[...])


@jax.jit
def _tc_mlp(packed, W1aT, W1bT, b1, W2T, b2, W3T, b3, WogT, WomT, bo):
    P, B = packed.shape
    blk = 2048
    grid = (B // blk,)

    def wspec(a):
        return pl.BlockSpec(a.shape, lambda i: (0,) * a.ndim)

    return pl.pallas_call(
        _tc_body,
        grid=grid,
        in_specs=[pl.BlockSpec((P, blk), lambda i: (0, i)),
                  wspec(W1aT), wspec(W1bT), wspec(b1), wspec(W2T), wspec(b2),
                  wspec(W3T), wspec(b3), wspec(WogT), wspec(WomT), wspec(bo)],
        out_specs=pl.BlockSpec((1, blk), lambda i: (0, i)),
        out_shape=jax.ShapeDtypeStruct((1, B), jnp.float32),
    )(packed, W1aT, W1bT, b1, W2T, b2, W3T, b3, WogT, WomT, bo)


def kernel(user_ids, item_ids, gmf_user, gmf_item, mlp_user, mlp_item,
           W1, b1, W2, b2, W3, b3, Wo, bo):
    B = user_ids.shape[0]
    V, D = gmf_user.shape
    uids = user_ids.astype(jnp.int32)
    iids = item_ids.astype(jnp.int32)
    rem = V % 128

    def edge(t):
        e = t.T[:, V - rem:] if rem else t.T[:, V - 128:]
        return jnp.pad(e, ((0, 0), (0, 128 - e.shape[1])))

    imgs = _sc_tile_image(D, V, gmf_user.T, gmf_item.T,
                          mlp_user.T, mlp_item.T,
                          edge(gmf_user), edge(gmf_item),
                          edge(mlp_user), edge(mlp_item))
    flats = [im.reshape(-1) for im in imgs]
    packed = _sc_gather_pack(B, D, V, uids, iids, *flats)
    out_t = _tc_mlp(packed, W1[:D].T, W1[D:].T, b1.reshape(-1, 1),
                    W2.T, b2.reshape(-1, 1), W3.T, b3.reshape(-1, 1),
                    Wo[:D].T, Wo[D:].T, bo.reshape(1, 1))
    return out_t.reshape(B, 1)


# speedup vs baseline: 3.4947x; 1.0120x over previous
import functools

import jax
import jax.numpy as jnp
from jax import lax
from jax.experimental import pallas as pl
from jax.experimental.pallas import tpu as pltpu
from jax.experimental.pallas import tpu_sc as plsc

NUM_CORES = 2
NUM_SUBCORES = 16
NW = NUM_CORES * NUM_SUBCORES
L = 16
SEG = 128
RW = 1024
NBUF = 8


@functools.partial(jax.jit, static_argnums=(0, 1))
def _sc_tile_image(D, V, gu_t, gi_t, mu_t, mi_t, gu_e, gi_e, mu_e, mi_e):
    ntc = (V + 127) // 128
    n_full = (V // RW)
    n_it = (n_full + NW - 1) // NW
    tail0 = n_full * RW
    ntr = (D + 7) // 8
    mesh = plsc.VectorSubcoreMesh(core_axis_name="c", subcore_axis_name="s")

    @functools.partial(
        pl.kernel,
        out_type=[jax.ShapeDtypeStruct((ntr * ntc * 8, 128), jnp.float32)] * 4,
        mesh=mesh,
        compiler_params=pltpu.CompilerParams(use_tc_tiling_on_sc=True),
        scratch_types=(
            [pltpu.VMEM((8, RW), jnp.float32)] * NBUF
            + [pltpu.SemaphoreType.DMA] * NBUF
            + [pltpu.SemaphoreType.DMA] * NBUF
            + [pltpu.VMEM((8, 128), jnp.float32), pltpu.SemaphoreType.DMA]
        ),
    )
    def copy_k(gu_h, gi_h, mu_h, mi_h, gu_eh, gi_eh, mu_eh, mi_eh,
               *rest):
        dsts = rest[:4]
        scratch = rest[4:]
        bufs = scratch[:NBUF]
        rsem = scratch[NBUF:2 * NBUF]
        wsem = scratch[2 * NBUF:3 * NBUF]
        tbuf, tsem = scratch[3 * NBUF], scratch[3 * NBUF + 1]
        srcs = (gu_h, gi_h, mu_h, mi_h)
        wid = lax.axis_index("s") * NUM_CORES + lax.axis_index("c")
        steps = [(t, tr) for t in range(4) for tr in range(ntr)]

        def body(it, carry):
            ch = it * NW + wid

            @pl.when(ch < n_full)
            def _active():
                off = pl.multiple_of(ch * RW, RW)

                def read(s, first_use):
                    t, tr = steps[s]
                    b = s % NBUF
                    drain = pltpu.make_async_copy(
                        srcs[0].at[pl.ds(0, 8), pl.ds(0, RW)],
                        bufs[b], wsem[b])
                    if first_use:
                        @pl.when(it > 0)
                        def _():
                            drain.wait()
                    else:
                        drain.wait()
                    return pltpu.async_copy(
                        srcs[t].at[pl.ds(tr * 8, 8), pl.ds(off, RW)],
                        bufs[b], rsem[b])

                handles = {s: read(s, True) for s in range(NBUF)}
                for s in range(len(steps)):
                    t, tr = steps[s]
                    b = s % NBUF
                    handles[s].wait()
                    r0 = pl.multiple_of(tr * ntc * 8 + (off // 16), 8)
                    for j in range(RW // 128):
                        pltpu.async_copy(
                            bufs[b].at[:, pl.ds(j * 128, 128)],
                            dsts[t].at[pl.ds(r0 + 8 * j, 8)],
                            wsem[b])
                    if s + NBUF < len(steps):
                        handles[s + NBUF] = read(s + NBUF, False)
            return carry

        lax.fori_loop(0, n_it, body, 0)

        for b in range(NBUF):
            pltpu.make_async_copy(srcs[0].at[pl.ds(0, 8), pl.ds(0, RW)],
                                  bufs[b], wsem[b]).wait()

        edge = (gu_eh, gi_eh, mu_eh, mi_eh)
        n_tail_full = (V - tail0) // 128
        n_items = n_tail_full + (1 if V % 128 else 0)
        items = [(t, tr, k) for t in range(4) for tr in range(ntr)
                 for k in range(n_items)]
        for n, (t, tr, k) in enumerate(items):
            @pl.when(wid == n % NW)
            def _tail(t=t, tr=tr, k=k):
                tc = tail0 // 128 + k
                if k < n_tail_full:
                    src = srcs[t].at[pl.ds(tr * 8, 8), pl.ds(tc * 128, 128)]
                else:
                    src = edge[t].at[pl.ds(tr * 8, 8), :]
                pltpu.async_copy(src, tbuf, tsem).wait()
                pltpu.async_copy(
                    tbuf, dsts[t].at[pl.ds((tr * ntc + tc) * 8, 8)],
                    tsem).wait()

    return copy_k(gu_t, gi_t, mu_t, mi_t, gu_e, gi_e, mu_e, mi_e)


@functools.partial(jax.jit, static_argnums=(0, 1, 2))
def _sc_gather_pack(B, D, V, uids, iids, *parts):
    b_per_w = B // NW
    n_el = b_per_w * D
    n_seg = n_el // SEG
    segs_per_d = b_per_w // SEG
    trow = ((V + 127) // 128) * 1024
    mesh = plsc.VectorSubcoreMesh(core_axis_name="c", subcore_axis_name="s")

    @functools.partial(
        pl.kernel,
        out_type=jax.ShapeDtypeStruct((3 * D, B), jnp.float32),
        mesh=mesh,
        compiler_params=pltpu.CompilerParams(use_tc_tiling_on_sc=False),
        scratch_types=[
            pltpu.VMEM((b_per_w,), jnp.int32),
            pltpu.VMEM((b_per_w,), jnp.int32),
            pltpu.VMEM((n_seg, SEG), jnp.int32),
            pltpu.VMEM((n_seg, SEG), jnp.int32),
            pltpu.VMEM((D, b_per_w), jnp.float32),
            pltpu.VMEM((D, b_per_w), jnp.float32),
            pltpu.VMEM((D, b_per_w), jnp.float32),
            pltpu.VMEM((D, b_per_w), jnp.float32),
            pltpu.SemaphoreType.DMA,
        ],
    )
    def gather(uids_h, iids_h, *args):
        gu_h, gi_h, mu_h, mi_h = args[:4]
        out_h = args[4]
        uidx, iidx, uel, iel, gub, gib, mub, mib, sem = args[5:]
        wid = lax.axis_index("s") * NUM_CORES + lax.axis_index("c")
        base = wid * b_per_w
        pltpu.sync_copy(uids_h.at[pl.ds(base, b_per_w)], uidx)
        pltpu.sync_copy(iids_h.at[pl.ds(base, b_per_w)], iidx)

        def build_idx(seg, _):
            d = seg // segs_per_d
            kd = (d // 8) * trow + (d % 8) * 128
            for g in range(SEG // L):
                s0 = (seg % segs_per_d) * SEG + g * L
                u = uidx[pl.ds(s0, L)]
                v = iidx[pl.ds(s0, L)]
                uel[seg, pl.ds(g * L, L)] = ((u >> 7) << 10) + (u & 127) + kd
                iel[seg, pl.ds(g * L, L)] = ((v >> 7) << 10) + (v & 127) + kd
            return _

        lax.fori_loop(0, n_seg, build_idx, 0)

        def fire(k, _):
            d = k
            for q in range(segs_per_d):
                seg = k * segs_per_d + q
                c0 = q * SEG
                pltpu.async_copy(gu_h.at[uel.at[seg]],
                                 gub.at[d, pl.ds(c0, SEG)], sem)
                pltpu.async_copy(gi_h.at[iel.at[seg]],
                                 gib.at[d, pl.ds(c0, SEG)], sem)
                pltpu.async_copy(mu_h.at[uel.at[seg]],
                                 mub.at[d, pl.ds(c0, SEG)], sem)
                pltpu.async_copy(mi_h.at[iel.at[seg]],
                                 mib.at[d, pl.ds(c0, SEG)], sem)
            return _

        lax.fori_loop(0, n_seg // segs_per_d, fire, 0)

        def drain(d, _):
            for buf in (gub, gib, mub, mib):
                pltpu.make_async_copy(gu_h.at[pl.ds(0, b_per_w)],
                                      buf.at[d], sem).wait()
            return _

        lax.fori_loop(0, D, drain, 0)

        def prod(d, _):
            for g in range(b_per_w // L):
                sl = pl.ds(g * L, L)
                gub[d, sl] = gub[d, sl] * gib[d, sl]
            return _

        lax.fori_loop(0, D, prod, 0)

        col = pl.ds(base, b_per_w)
        pltpu.sync_copy(gub, out_h.at[pl.ds(0, D), col])
        pltpu.sync_copy(mub, out_h.at[pl.ds(D, D), col])
        pltpu.sync_copy(mib, out_h.at[pl.ds(2 * D, D), col])

    return gather(uids, iids, *parts)


def _tc_body(x_r, w1a_r, w1b_r, b1_r, w2_r, b2_r, w3_r, b3_r,
             wog_r, wom_r, bo_r, out_r):
    f32 = jnp.float32
    x = x_r[...]
    prod, mu, mi = x[0:32, :], x[32:64, :], x[64:96, :]
    h = (jnp.dot(w1a_r[...], mu, preferred_element_type=f32)
         + jnp.dot(w1b_r[...], mi, preferred_element_type=f32)
         + b1_r[...])
    h = jnp.where(h >= 0, h, 0.01 * h)
    h = jnp.dot(w2_r[...], h, preferred_element_type=f32) + b2_r[...]
    h = jnp.where(h >= 0, h, 0.01 * h)
    h = jnp.dot(w3_r[...], h, preferred_element_type=f32) + b3_r[...]
    out_r[...] = (jnp.dot(wog_r[...], prod, preferred_element_type=f32)
                  + jnp.dot(wom_r[...], h, preferred_element_type=f32)
                  + bo_r[...])


@jax.jit
def _tc_mlp(packed, W1aT, W1bT, b1, W2T, b2, W3T, b3, WogT, WomT, bo):
    P, B = packed.shape
    blk = 2048
    grid = (B // blk,)

    def wspec(a):
        return pl.BlockSpec(a.shape, lambda i: (0,) * a.ndim)

    return pl.pallas_call(
        _tc_body,
        grid=grid,
        in_specs=[pl.BlockSpec((P, blk), lambda i: (0, i)),
                  wspec(W1aT), wspec(W1bT), wspec(b1), wspec(W2T), wspec(b2),
                  wspec(W3T), wspec(b3), wspec(WogT), wspec(WomT), wspec(bo)],
        out_specs=pl.BlockSpec((1, blk), lambda i: (0, i)),
        out_shape=jax.ShapeDtypeStruct((1, B), jnp.float32),
    )(packed, W1aT, W1bT, b1, W2T, b2, W3T, b3, WogT, WomT, bo)


def kernel(user_ids, item_ids, gmf_user, gmf_item, mlp_user, mlp_item,
           W1, b1, W2, b2, W3, b3, Wo, bo):
    B = user_ids.shape[0]
    V, D = gmf_user.shape
    uids = user_ids.astype(jnp.int32)
    iids = item_ids.astype(jnp.int32)
    rem = V % 128

    def edge(t):
        e = t.T[:, V - rem:] if rem else t.T[:, V - 128:]
        return jnp.pad(e, ((0, 0), (0, 128 - e.shape[1])))

    imgs = _sc_tile_image(D, V, gmf_user.T, gmf_item.T,
                          mlp_user.T, mlp_item.T,
                          edge(gmf_user), edge(gmf_item),
                          edge(mlp_user), edge(mlp_item))
    flats = [im.reshape(-1) for im in imgs]
    packed = _sc_gather_pack(B, D, V, uids, iids, *flats)
    out_t = _tc_mlp(packed, W1[:D].T, W1[D:].T, b1.reshape(-1, 1),
                    W2.T, b2.reshape(-1, 1), W3.T, b3.reshape(-1, 1),
                    Wo[:D].T, Wo[D:].T, bo.reshape(1, 1))
    return out_t.reshape(B, 1)
